# baseline (speedup 1.0000x reference)
import functools

import jax
import jax.numpy as jnp
from jax import lax
from jax.experimental import pallas as pl
from jax.experimental.pallas import tpu as pltpu
from jax.experimental.pallas import tpu_sc as plsc

_LN_EPS = 1e-12


def _sc_gather(table, idx):
    V, D = table.shape
    N = idx.shape[0]
    info = plsc.get_sparse_core_info()
    NC, NS = info.num_cores, info.num_subcores
    NW = NC * NS
    assert N % NW == 0
    T = N // NW
    CH = 32
    NCH = T // CH
    assert NCH >= 4
    mesh = plsc.VectorSubcoreMesh(core_axis_name="c", subcore_axis_name="s")

    @functools.partial(
        pl.kernel,
        mesh=mesh,
        out_type=jax.ShapeDtypeStruct((N, D), jnp.float32),
        compiler_params=pltpu.CompilerParams(needs_layout_passes=False),
        scratch_types=[
            pltpu.VMEM((T,), jnp.int32),
            pltpu.VMEM((4, CH, D), jnp.float32),
            pltpu.SemaphoreType.DMA,
            pltpu.SemaphoreType.DMA,
            pltpu.SemaphoreType.DMA,
            pltpu.SemaphoreType.DMA,
            pltpu.SemaphoreType.DMA,
            pltpu.SemaphoreType.DMA,
            pltpu.SemaphoreType.DMA,
            pltpu.SemaphoreType.DMA,
        ],
    )
    def k(table_hbm, idx_hbm, out_hbm, idx_v, rows,
          semg0, semg1, semg2, semg3, semo0, semo1, semo2, semo3):
        wid = lax.axis_index("s") * NC + lax.axis_index("c")
        base = wid * T
        semg = (semg0, semg1, semg2, semg3)
        semo = (semo0, semo1, semo2, semo3)

        pltpu.sync_copy(idx_hbm.at[pl.ds(base, T)], idx_v)

        def g_descr(j, s):
            return pltpu.make_async_copy(
                table_hbm.at[idx_v.at[pl.ds(j * CH, CH)]], rows.at[s],
                semg[s])

        def o_descr(j, s):
            return pltpu.make_async_copy(
                rows.at[s], out_hbm.at[pl.ds(base + j * CH, CH)], semo[s])

        def chunk(j, s, so):
            g_descr(j, s).wait()

            @pl.when(j >= 2)
            def _():
                o_descr(j - 2, so).wait()

            @pl.when(j < NCH - 2)
            def _():
                g_descr(j + 2, so).start()

            o_descr(j, s).start()

        g_descr(0, 0).start()
        g_descr(1, 1).start()

        @pl.loop(0, NCH, step=4)
        def _ring(b):
            for q in range(4):
                chunk(b + q, q, (q + 2) % 4)

        o_descr(NCH - 2, (NCH - 2) % 4).wait()
        o_descr(NCH - 1, (NCH - 1) % 4).wait()

    return k(table, idx)


def _tc_fused_ln(feat2, pos_rows, sidf, seg_table, gamma2, beta2):
    N, D = feat2.shape
    BT = 1024

    def body(f_ref, p_ref, sid_ref, seg_ref, g_ref, b_ref, o_ref):
        x = f_ref[...] + p_ref[...]
        seg0 = seg_ref[0:1, :]
        dseg = seg_ref[1:2, :] - seg0
        x = x + seg0 + sid_ref[...] * dseg
        mean = jnp.mean(x, axis=1, keepdims=True)
        xc = x - mean
        var = jnp.mean(xc * xc, axis=1, keepdims=True)
        rstd = lax.rsqrt(var + _LN_EPS)
        o_ref[...] = xc * rstd * g_ref[...] + b_ref[...]

    return pl.pallas_call(
        body,
        grid=(N // BT,),
        in_specs=[
            pl.BlockSpec((BT, D), lambda i: (i, 0)),
            pl.BlockSpec((BT, D), lambda i: (i, 0)),
            pl.BlockSpec((BT, 1), lambda i: (i, 0)),
            pl.BlockSpec((2, D), lambda i: (0, 0)),
            pl.BlockSpec((1, D), lambda i: (0, 0)),
            pl.BlockSpec((1, D), lambda i: (0, 0)),
        ],
        out_specs=pl.BlockSpec((BT, D), lambda i: (i, 0)),
        out_shape=jax.ShapeDtypeStruct((N, D), jnp.float32),
        compiler_params=pltpu.CompilerParams(
            dimension_semantics=("arbitrary",)),
    )(feat2, pos_rows, sidf, seg_table, gamma2, beta2)


def kernel(feat_embs, position_ids, segment_ids, pos_table, seg_table,
           ln_gamma, ln_beta):
    B, S, D = feat_embs.shape
    N = B * S
    feat2 = feat_embs.reshape(N, D)
    pos = position_ids.reshape(N).astype(jnp.int32)
    sidf = segment_ids.reshape(N, 1).astype(jnp.float32)
    seg32 = seg_table.astype(jnp.float32)
    g2 = ln_gamma.reshape(1, D)
    b2 = ln_beta.reshape(1, D)
    H = N // 2
    pr0 = _sc_gather(pos_table, pos[:H])
    pr1 = _sc_gather(pos_table, pos[H:])
    o0 = _tc_fused_ln(feat2[:H], pr0, sidf[:H], seg32, g2, b2)
    o1 = _tc_fused_ln(feat2[H:], pr1, sidf[H:], seg32, g2, b2)
    return jnp.concatenate([o0, o1], axis=0).reshape(B, S, D)


# speedup vs baseline: 1.6725x; 1.6725x over previous
import functools

import jax
import jax.numpy as jnp
from jax import lax
from jax.experimental import pallas as pl
from jax.experimental.pallas import tpu as pltpu
from jax.experimental.pallas import tpu_sc as plsc

_LN_EPS = 1e-12


def _sc_gather(table, idx):
    V, D = table.shape
    N = idx.shape[0]
    info = plsc.get_sparse_core_info()
    NC, NS = info.num_cores, info.num_subcores
    NW = NC * NS
    assert N % NW == 0
    T = N // NW
    CH = 32
    NCH = T // CH
    assert NCH >= 4
    mesh = plsc.VectorSubcoreMesh(core_axis_name="c", subcore_axis_name="s")

    @functools.partial(
        pl.kernel,
        mesh=mesh,
        out_type=jax.ShapeDtypeStruct((N, D), jnp.float32),
        compiler_params=pltpu.CompilerParams(needs_layout_passes=False),
        scratch_types=[
            pltpu.VMEM((T,), jnp.int32),
            pltpu.VMEM((4, CH, D), jnp.float32),
            pltpu.SemaphoreType.DMA,
            pltpu.SemaphoreType.DMA,
            pltpu.SemaphoreType.DMA,
            pltpu.SemaphoreType.DMA,
            pltpu.SemaphoreType.DMA,
            pltpu.SemaphoreType.DMA,
            pltpu.SemaphoreType.DMA,
            pltpu.SemaphoreType.DMA,
        ],
    )
    def k(table_hbm, idx_hbm, out_hbm, idx_v, rows,
          semg0, semg1, semg2, semg3, semo0, semo1, semo2, semo3):
        wid = lax.axis_index("s") * NC + lax.axis_index("c")
        base = wid * T
        semg = (semg0, semg1, semg2, semg3)
        semo = (semo0, semo1, semo2, semo3)

        pltpu.sync_copy(idx_hbm.at[pl.ds(base, T)], idx_v)

        def g_descr(j, s):
            return pltpu.make_async_copy(
                table_hbm.at[idx_v.at[pl.ds(j * CH, CH)]], rows.at[s],
                semg[s])

        def o_descr(j, s):
            return pltpu.make_async_copy(
                rows.at[s], out_hbm.at[pl.ds(base + j * CH, CH)], semo[s])

        def chunk(j, s, so):
            g_descr(j, s).wait()

            @pl.when(j >= 2)
            def _():
                o_descr(j - 2, so).wait()

            @pl.when(j < NCH - 2)
            def _():
                g_descr(j + 2, so).start()

            o_descr(j, s).start()

        g_descr(0, 0).start()
        g_descr(1, 1).start()

        @pl.loop(0, NCH, step=4)
        def _ring(b):
            for q in range(4):
                chunk(b + q, q, (q + 2) % 4)

        o_descr(NCH - 2, (NCH - 2) % 4).wait()
        o_descr(NCH - 1, (NCH - 1) % 4).wait()

    return k(table, idx)


def _tc_fused_ln(feat2, pos_rows, sidf, seg_table, gamma2, beta2):
    N, D = feat2.shape
    BT = 2048

    def body(f_ref, p_ref, sid_ref, seg_ref, g_ref, b_ref, o_ref):
        x = f_ref[...] + p_ref[...]
        seg0 = seg_ref[0:1, :]
        dseg = seg_ref[1:2, :] - seg0
        x = x + seg0 + sid_ref[...] * dseg
        mean = jnp.mean(x, axis=1, keepdims=True)
        xc = x - mean
        var = jnp.mean(xc * xc, axis=1, keepdims=True)
        rstd = lax.rsqrt(var + _LN_EPS)
        o_ref[...] = xc * rstd * g_ref[...] + b_ref[...]

    return pl.pallas_call(
        body,
        grid=(N // BT,),
        in_specs=[
            pl.BlockSpec((BT, D), lambda i: (i, 0)),
            pl.BlockSpec((BT, D), lambda i: (i, 0)),
            pl.BlockSpec((BT, 1), lambda i: (i, 0)),
            pl.BlockSpec((2, D), lambda i: (0, 0)),
            pl.BlockSpec((1, D), lambda i: (0, 0)),
            pl.BlockSpec((1, D), lambda i: (0, 0)),
        ],
        out_specs=pl.BlockSpec((BT, D), lambda i: (i, 0)),
        out_shape=jax.ShapeDtypeStruct((N, D), jnp.float32),
        compiler_params=pltpu.CompilerParams(
            dimension_semantics=("arbitrary",)),
    )(feat2, pos_rows, sidf, seg_table, gamma2, beta2)


def kernel(feat_embs, position_ids, segment_ids, pos_table, seg_table,
           ln_gamma, ln_beta):
    B, S, D = feat_embs.shape
    N = B * S
    feat2 = feat_embs.reshape(N, D)
    pos = position_ids.reshape(N).astype(jnp.int32)
    sidf = segment_ids.reshape(N, 1).astype(jnp.float32)
    pos_rows = _sc_gather(pos_table, pos)
    out2 = _tc_fused_ln(feat2, pos_rows, sidf, seg_table.astype(jnp.float32),
                        ln_gamma.reshape(1, D), ln_beta.reshape(1, D))
    return out2.reshape(B, S, D)


# speedup vs baseline: 1.6747x; 1.0013x over previous
import functools

import jax
import jax.numpy as jnp
from jax import lax
from jax.experimental import pallas as pl
from jax.experimental.pallas import tpu as pltpu
from jax.experimental.pallas import tpu_sc as plsc

_LN_EPS = 1e-12


def _sc_gather(table, idx):
    V, D = table.shape
    N = idx.shape[0]
    info = plsc.get_sparse_core_info()
    NC, NS = info.num_cores, info.num_subcores
    NW = NC * NS
    assert N % NW == 0
    T = N // NW
    CH = 32
    NCH = T // CH
    assert NCH >= 4 and NCH % 4 == 0
    mesh = plsc.VectorSubcoreMesh(core_axis_name="c", subcore_axis_name="s")

    @functools.partial(
        pl.kernel,
        mesh=mesh,
        out_type=jax.ShapeDtypeStruct((N, D), jnp.float32),
        compiler_params=pltpu.CompilerParams(needs_layout_passes=False),
        scratch_types=[
            pltpu.VMEM((T,), jnp.int32),
            pltpu.VMEM((4, CH, D), jnp.float32),
            pltpu.SemaphoreType.DMA,
            pltpu.SemaphoreType.DMA,
            pltpu.SemaphoreType.DMA,
            pltpu.SemaphoreType.DMA,
            pltpu.SemaphoreType.DMA,
            pltpu.SemaphoreType.DMA,
            pltpu.SemaphoreType.DMA,
            pltpu.SemaphoreType.DMA,
        ],
    )
    def k(table_hbm, idx_hbm, out_hbm, idx_v, rows,
          semg0, semg1, semg2, semg3, semo0, semo1, semo2, semo3):
        wid = lax.axis_index("s") * NC + lax.axis_index("c")
        base = wid * T
        semg = (semg0, semg1, semg2, semg3)
        semo = (semo0, semo1, semo2, semo3)

        pltpu.sync_copy(idx_hbm.at[pl.ds(base, T)], idx_v)

        def g_descr(j, s):
            return pltpu.make_async_copy(
                table_hbm.at[idx_v.at[pl.ds(j * CH, CH)]], rows.at[s],
                semg[s])

        def o_descr(j, s):
            return pltpu.make_async_copy(
                rows.at[s], out_hbm.at[pl.ds(base + j * CH, CH)], semo[s])

        def chunk(j, s, so):
            g_descr(j, s).wait()

            @pl.when(j >= 2)
            def _():
                o_descr(j - 2, so).wait()

            @pl.when(j < NCH - 2)
            def _():
                g_descr(j + 2, so).start()

            o_descr(j, s).start()

        g_descr(0, 0).start()
        g_descr(1, 1).start()

        @pl.loop(0, NCH, step=4)
        def _ring(b):
            for q in range(4):
                chunk(b + q, q, (q + 2) % 4)

        o_descr(NCH - 2, (NCH - 2) % 4).wait()
        o_descr(NCH - 1, (NCH - 1) % 4).wait()

    return k(table, idx)


def _tc_fused_ln(feat2, pos_rows, sidf, seg_table, gamma2, beta2):
    N, D = feat2.shape
    BT = 2048

    def body(f_ref, p_ref, sid_ref, seg_ref, g_ref, b_ref, o_ref):
        x = f_ref[...] + p_ref[...]
        seg0 = seg_ref[0:1, :]
        dseg = seg_ref[1:2, :] - seg0
        x = x + seg0 + sid_ref[...] * dseg
        mean = jnp.mean(x, axis=1, keepdims=True)
        xc = x - mean
        var = jnp.mean(xc * xc, axis=1, keepdims=True)
        rstd = lax.rsqrt(var + _LN_EPS)
        o_ref[...] = xc * rstd * g_ref[...] + b_ref[...]

    return pl.pallas_call(
        body,
        grid=(N // BT,),
        in_specs=[
            pl.BlockSpec((BT, D), lambda i: (i, 0)),
            pl.BlockSpec((BT, D), lambda i: (i, 0)),
            pl.BlockSpec((BT, 1), lambda i: (i, 0)),
            pl.BlockSpec((2, D), lambda i: (0, 0)),
            pl.BlockSpec((1, D), lambda i: (0, 0)),
            pl.BlockSpec((1, D), lambda i: (0, 0)),
        ],
        out_specs=pl.BlockSpec((BT, D), lambda i: (i, 0)),
        out_shape=jax.ShapeDtypeStruct((N, D), jnp.float32),
        compiler_params=pltpu.CompilerParams(
            dimension_semantics=("arbitrary",)),
    )(feat2, pos_rows, sidf, seg_table, gamma2, beta2)


def kernel(feat_embs, position_ids, segment_ids, pos_table, seg_table,
           ln_gamma, ln_beta):
    B, S, D = feat_embs.shape
    N = B * S
    feat2 = feat_embs.reshape(N, D)
    pos = position_ids.reshape(N).astype(jnp.int32)
    sidf = segment_ids.reshape(N, 1).astype(jnp.float32)
    pos_rows = _sc_gather(pos_table, pos)
    out2 = _tc_fused_ln(feat2, pos_rows, sidf, seg_table.astype(jnp.float32),
                        ln_gamma.reshape(1, D), ln_beta.reshape(1, D))
    return out2.reshape(B, S, D)
